# baseline (speedup 1.0000x reference)
import functools

import jax
import jax.numpy as jnp
from jax import lax
from jax.experimental import pallas as pl
from jax.experimental.pallas import tpu as pltpu
from jax.experimental.pallas import tpu_sc as plsc

CUTOFF = 5.0
N_NODES = 10000
N_EDGES = 320000
HIDDEN = 128
N_RBF = 64

NC, NS = 2, 16
NW = NC * NS
K = 48
CHUNKS = 210
E_PAD = NW * K * CHUNKS
N_PAD = 10240
ROWS_PER_TILE = N_PAD // NS

TBL = 16384
INV_H = TBL / CUTOFF
TG_PAD = 18432


def _filter_body(d_ref, c_ref, g_ref, w1_ref, b1_ref, w2_ref, b2_ref, o_ref):
    d = d_ref[...]
    g = g_ref[0, 0]
    diff = d - c_ref[...]
    rbf = jnp.exp(-g * diff * diff)
    h = jnp.dot(rbf, w1_ref[...], preferred_element_type=jnp.float32) + b1_ref[...]
    h = h * jax.nn.sigmoid(h)
    w = jnp.dot(h, w2_ref[...], preferred_element_type=jnp.float32) + b2_ref[...]
    xc = jnp.clip(d * (1.0 / CUTOFF), 0.0, 1.0)
    cc = 0.5 * (jnp.cos(jnp.pi * xc) + 1.0) * (xc < 1.0).astype(jnp.float32)
    o_ref[...] = w * cc


def _table_call(dgrid, centers, gamma, W1, b1, W2, b2):
    BE = 2048
    return pl.pallas_call(
        _filter_body,
        grid=(TG_PAD // BE,),
        in_specs=[
            pl.BlockSpec((BE, 1), lambda i: (i, 0)),
            pl.BlockSpec((1, N_RBF), lambda i: (0, 0)),
            pl.BlockSpec(memory_space=pltpu.SMEM),
            pl.BlockSpec((N_RBF, HIDDEN), lambda i: (0, 0)),
            pl.BlockSpec((1, HIDDEN), lambda i: (0, 0)),
            pl.BlockSpec((HIDDEN, HIDDEN), lambda i: (0, 0)),
            pl.BlockSpec((1, HIDDEN), lambda i: (0, 0)),
        ],
        out_specs=pl.BlockSpec((BE, HIDDEN), lambda i: (i, 0)),
        out_shape=jax.ShapeDtypeStruct((TG_PAD, HIDDEN), jnp.float32),
    )(
        dgrid.reshape(TG_PAD, 1),
        centers.reshape(1, N_RBF),
        gamma.reshape(1, 1),
        W1,
        b1.reshape(1, HIDDEN),
        W2,
        b2.reshape(1, HIDDEN),
    )


def _xd_body(x_ref, wd_ref, o_ref):
    o_ref[...] = jnp.dot(x_ref[...], wd_ref[...],
                         preferred_element_type=jnp.float32)


def _xd_call(x, Wd):
    BN = 2000
    return pl.pallas_call(
        _xd_body,
        grid=(N_NODES // BN,),
        in_specs=[
            pl.BlockSpec((BN, HIDDEN), lambda i: (i, 0)),
            pl.BlockSpec((HIDDEN, HIDDEN), lambda i: (0, 0)),
        ],
        out_specs=pl.BlockSpec((BN, HIDDEN), lambda i: (i, 0)),
        out_shape=jax.ShapeDtypeStruct((N_NODES, HIDDEN), jnp.float32),
    )(x, Wd)


def _combine_body(a_ref, b_ref, o_ref):
    o_ref[...] = a_ref[...] + b_ref[...]


def _combine_call(p0, p1):
    BN = 2000
    return pl.pallas_call(
        _combine_body,
        grid=(N_NODES // BN,),
        in_specs=[
            pl.BlockSpec((BN, HIDDEN), lambda i: (i, 0)),
            pl.BlockSpec((BN, HIDDEN), lambda i: (i, 0)),
        ],
        out_specs=pl.BlockSpec((BN, HIDDEN), lambda i: (i, 0)),
        out_shape=jax.ShapeDtypeStruct((N_NODES, HIDDEN), jnp.float32),
    )(p0, p1)


def _sc_body(xd_h, p_h, d_h, src_h, dst_h, out_h,
             idx_s0, idx_d0, idx_t0, db0, wf0, rows0, t010,
             idx_s1, idx_d1, idx_t1, db1, wf1, rows1, t011,
             acc, sem0, sem1):
    c = lax.axis_index("c")
    s = lax.axis_index("s")
    wid = c * NS + s
    base_w = wid * CHUNKS * K

    @plsc.parallel_loop(0, K)
    def _zrow(i):
        for j in range(HIDDEN // 16):
            rows0[i, pl.ds(j * 16, 16)] = jnp.zeros((16,), jnp.float32)

    zbase = s * ROWS_PER_TILE
    n_full = ROWS_PER_TILE // K
    z_rem = ROWS_PER_TILE - n_full * K
    for t in range(n_full):
        pltpu.sync_copy(rows0, acc.at[pl.ds(zbase + t * K, K)])
    if z_rem:
        pltpu.sync_copy(rows0.at[pl.ds(0, z_rem)],
                        acc.at[pl.ds(zbase + n_full * K, z_rem)])
    plsc.subcore_barrier()

    def _start(ci, idx_s, idx_d, idx_t, db, wf, rows, t01, sem):
        base = base_w + ci * K
        pltpu.sync_copy(src_h.at[pl.ds(base, K)], idx_s)
        pltpu.sync_copy(dst_h.at[pl.ds(base, K)], idx_d)
        pltpu.sync_copy(d_h.at[pl.ds(base, K)], db)
        for t in range(K // 16):
            sl = pl.ds(t * 16, 16)
            fi = db[sl] * INV_H
            ii = jnp.minimum(fi.astype(jnp.int32), TBL - 1)
            idx_t[sl] = ii
            wf[sl] = fi - ii.astype(jnp.float32)
        pltpu.async_copy(xd_h.at[idx_s], rows, sem)
        pltpu.async_copy(p_h.at[idx_t], t01, sem)

    def _finish(idx_s, idx_d, idx_t, db, wf, rows, t01, sem):
        pltpu.make_async_copy(xd_h.at[idx_s], rows, sem).wait()
        pltpu.make_async_copy(p_h.at[idx_t], t01, sem).wait()

        @plsc.parallel_loop(0, K // 16)
        def _mulgrp(g):
            wvec = wf[pl.ds(g * 16, 16)]
            for r in range(16):
                i = g * 16 + r
                w = wvec[r]
                for j in range(HIDDEN // 16):
                    sl = pl.ds(j * 16, 16)
                    t0 = t01[i, sl]
                    t1 = t01[i, pl.ds(HIDDEN + j * 16, 16)]
                    rows[i, sl] = rows[i, sl] * (t0 + w * (t1 - t0))

        pltpu.sync_copy(rows, acc.at[idx_d], add=True)

    buf0 = (idx_s0, idx_d0, idx_t0, db0, wf0, rows0, t010, sem0)
    buf1 = (idx_s1, idx_d1, idx_t1, db1, wf1, rows1, t011, sem1)
    _start(0, *buf0)

    def _pair(j, carry):
        _start(2 * j + 1, *buf1)
        _finish(*buf0)

        @pl.when(j < CHUNKS // 2 - 1)
        def _():
            _start(2 * j + 2, *buf0)
        _finish(*buf1)
        return carry
    lax.fori_loop(0, CHUNKS // 2, _pair, 0)
    plsc.subcore_barrier()

    for t in range(n_full):
        pltpu.sync_copy(acc.at[pl.ds(zbase + t * K, K)], rows0)
        pltpu.sync_copy(rows0, out_h.at[c, pl.ds(zbase + t * K, K)])
    if z_rem:
        pltpu.sync_copy(acc.at[pl.ds(zbase + n_full * K, z_rem)],
                        rows0.at[pl.ds(0, z_rem)])
        pltpu.sync_copy(rows0.at[pl.ds(0, z_rem)],
                        out_h.at[c, pl.ds(zbase + n_full * K, z_rem)])


def _sc_scratch():
    per_buf = [
        pltpu.VMEM((K,), jnp.int32),
        pltpu.VMEM((K,), jnp.int32),
        pltpu.VMEM((K,), jnp.int32),
        pltpu.VMEM((K,), jnp.float32),
        pltpu.VMEM((K,), jnp.float32),
        pltpu.VMEM((K, HIDDEN), jnp.float32),
        pltpu.VMEM((K, 2 * HIDDEN), jnp.float32),
    ]
    return (per_buf + per_buf
            + [pltpu.VMEM_SHARED((N_PAD, HIDDEN), jnp.float32),
               pltpu.SemaphoreType.DMA,
               pltpu.SemaphoreType.DMA])


_sc_call = functools.partial(
    pl.kernel,
    out_type=jax.ShapeDtypeStruct((NC, N_PAD, HIDDEN), jnp.float32),
    mesh=plsc.VectorSubcoreMesh(core_axis_name="c", subcore_axis_name="s"),
    scratch_types=_sc_scratch(),
)(_sc_body)


def kernel(x, edge_index, distances, centers, gamma, W1, b1, W2, b2, Wd):
    src = edge_index[0].astype(jnp.int32)
    dst = edge_index[1].astype(jnp.int32)
    pad = E_PAD - N_EDGES
    dist_pad = jnp.concatenate(
        [distances, jnp.full((pad,), CUTOFF, jnp.float32)])
    src_p = jnp.concatenate([src, jnp.zeros((pad,), jnp.int32)])
    dst_p = jnp.concatenate([dst, jnp.zeros((pad,), jnp.int32)])

    dgrid = jnp.arange(TG_PAD, dtype=jnp.float32) * (CUTOFF / TBL)
    table = _table_call(dgrid, centers.astype(jnp.float32),
                        gamma.astype(jnp.float32), W1, b1, W2, b2)
    pairs = jnp.concatenate([table[:TBL], table[1:TBL + 1]], axis=1)
    xd = _xd_call(x, Wd)
    parts = _sc_call(xd, pairs, dist_pad, src_p, dst_p)
    return _combine_call(parts[0], parts[1])
